# baseline (speedup 1.0000x reference)
import functools

import jax
import jax.numpy as jnp
import numpy as np
from jax.experimental import pallas as pl
from jax.experimental.pallas import tpu as pltpu

_B, _IN, _HID, _OUT, _T = 32, 2048, 1024, 512, 300
_THETA = 10.0
_ALPHA_SR = float(np.exp(-1.0 / 10.0))
_ALPHA_REF = float(np.exp(-1.0 / 2.0))
_REF_SCALE = 2.0 * _THETA


def _layer_body(x_ref, w_ref, o_ref, u_ref, r_ref, z_ref, *, tc, bh):
    t_idx = pl.program_id(2)

    @pl.when(t_idx == 0)
    def _():
        u_ref[...] = jnp.zeros_like(u_ref)
        r_ref[...] = jnp.zeros_like(r_ref)

    cin = x_ref.shape[-1]
    x = x_ref[...].reshape(tc * bh, cin)
    z_ref[...] = jnp.dot(x, w_ref[...], preferred_element_type=jnp.float32)

    u = u_ref[...]
    r = r_ref[...]
    for t in range(tc):
        zt = z_ref[t * bh:(t + 1) * bh, :]
        u = _ALPHA_SR * u + zt
        m = u + r
        s = (m >= _THETA).astype(jnp.float32)
        o_ref[t] = s.astype(o_ref.dtype)
        r = _ALPHA_REF * r - _REF_SCALE * s
    u_ref[...] = u
    r_ref[...] = r


def _slayer_layer_pallas(x_tbc, w_t, *, bo, tc):
    t_dim, b, cin = x_tbc.shape
    cout = w_t.shape[1]
    bh = b // 2
    grid = (2, cout // bo, t_dim // tc)
    return pl.pallas_call(
        functools.partial(_layer_body, tc=tc, bh=bh),
        grid=grid,
        in_specs=[
            pl.BlockSpec((tc, bh, cin), lambda i, j, k: (k, i, 0)),
            pl.BlockSpec((cin, bo), lambda i, j, k: (0, j)),
        ],
        out_specs=pl.BlockSpec((tc, bh, bo), lambda i, j, k: (k, i, j)),
        out_shape=jax.ShapeDtypeStruct((t_dim, b, cout), jnp.bfloat16),
        scratch_shapes=[
            pltpu.VMEM((bh, bo), jnp.float32),
            pltpu.VMEM((bh, bo), jnp.float32),
            pltpu.VMEM((tc * bh, bo), jnp.float32),
        ],
        compiler_params=pltpu.CompilerParams(
            dimension_semantics=("parallel", "parallel", "arbitrary"),
            vmem_limit_bytes=56 * 1024 * 1024,
        ),
        name="slayer_layer",
    )(x_tbc, w_t)


def kernel(spike_input, W1, W2):
    x = jnp.transpose(spike_input.astype(jnp.bfloat16), (2, 0, 1))
    s1 = _slayer_layer_pallas(x, W1.T.astype(jnp.bfloat16),
                              bo=1024, tc=50)
    s2 = _slayer_layer_pallas(s1, W2.T.astype(jnp.bfloat16),
                              bo=512, tc=50)
    return jnp.transpose(s2, (1, 2, 0)).astype(jnp.float32)


# speedup vs baseline: 1.0503x; 1.0503x over previous
import functools

import jax
import jax.numpy as jnp
import numpy as np
from jax.experimental import pallas as pl
from jax.experimental.pallas import tpu as pltpu

_B, _IN, _HID, _OUT, _T = 32, 2048, 1024, 512, 300
_THETA = 10.0
_ALPHA_SR = float(np.exp(-1.0 / 10.0))
_ALPHA_REF = float(np.exp(-1.0 / 2.0))
_REF_SCALE = 2.0 * _THETA


def _layer_body(x_ref, w_ref, o_ref, u_ref, r_ref, z_ref, *, tc, bh):
    t_idx = pl.program_id(2)

    @pl.when(t_idx == 0)
    def _():
        u_ref[...] = jnp.zeros_like(u_ref)
        r_ref[...] = jnp.zeros_like(r_ref)

    cin = x_ref.shape[-1]
    x = x_ref[...].reshape(tc * bh, cin).astype(jnp.bfloat16)
    z_ref[...] = jnp.dot(x, w_ref[...], preferred_element_type=jnp.float32)

    u = u_ref[...]
    r = r_ref[...]
    for t in range(tc):
        zt = z_ref[t * bh:(t + 1) * bh, :]
        u = _ALPHA_SR * u + zt
        m = u + r
        s = (m >= _THETA).astype(jnp.float32)
        o_ref[t] = s.astype(o_ref.dtype)
        r = _ALPHA_REF * r - _REF_SCALE * s
    u_ref[...] = u
    r_ref[...] = r


def _slayer_layer_pallas(x_tbc, w_t, *, bo, tc, nb=2):
    t_dim, b, cin = x_tbc.shape
    cout = w_t.shape[1]
    bh = b // nb
    grid = (nb, cout // bo, t_dim // tc)
    return pl.pallas_call(
        functools.partial(_layer_body, tc=tc, bh=bh),
        grid=grid,
        in_specs=[
            pl.BlockSpec((tc, bh, cin), lambda i, j, k: (k, i, 0)),
            pl.BlockSpec((cin, bo), lambda i, j, k: (0, j)),
        ],
        out_specs=pl.BlockSpec((tc, bh, bo), lambda i, j, k: (k, i, j)),
        out_shape=jax.ShapeDtypeStruct((t_dim, b, cout), jnp.bfloat16),
        scratch_shapes=[
            pltpu.VMEM((bh, bo), jnp.float32),
            pltpu.VMEM((bh, bo), jnp.float32),
            pltpu.VMEM((tc * bh, bo), jnp.float32),
        ],
        compiler_params=pltpu.CompilerParams(
            dimension_semantics=("parallel", "parallel", "arbitrary"),
            vmem_limit_bytes=56 * 1024 * 1024,
        ),
        name="slayer_layer",
    )(x_tbc, w_t)


def kernel(spike_input, W1, W2):
    x = jnp.transpose(spike_input.astype(jnp.int8), (2, 0, 1))
    s1 = _slayer_layer_pallas(x, W1.T.astype(jnp.bfloat16),
                              bo=1024, tc=25, nb=1)
    s2 = _slayer_layer_pallas(s1, W2.T.astype(jnp.bfloat16),
                              bo=512, tc=50)
    return jnp.transpose(s2, (1, 2, 0)).astype(jnp.float32)


# speedup vs baseline: 1.0601x; 1.0093x over previous
import functools

import jax
import jax.numpy as jnp
import numpy as np
from jax.experimental import pallas as pl
from jax.experimental.pallas import tpu as pltpu

_B, _IN, _HID, _OUT, _T = 32, 2048, 1024, 512, 300
_THETA = 10.0
_ALPHA_SR = float(np.exp(-1.0 / 10.0))
_ALPHA_REF = float(np.exp(-1.0 / 2.0))
_REF_SCALE = 2.0 * _THETA


def _layer_body(x_ref, w_ref, o_ref, u_ref, r_ref, z_ref, *, tc, bh):
    t_idx = pl.program_id(2)

    @pl.when(t_idx == 0)
    def _():
        u_ref[...] = jnp.zeros_like(u_ref)
        r_ref[...] = jnp.zeros_like(r_ref)

    cin = x_ref.shape[-1]
    x = x_ref[...].reshape(tc * bh, cin).astype(jnp.bfloat16)
    z_ref[...] = jnp.dot(x, w_ref[...], preferred_element_type=jnp.float32)

    u = u_ref[...]
    r = r_ref[...]
    for t in range(tc):
        zt = z_ref[t * bh:(t + 1) * bh, :]
        u = _ALPHA_SR * u + zt
        m = u + r
        s = (m >= _THETA).astype(jnp.float32)
        o_ref[t] = s.astype(o_ref.dtype)
        r = _ALPHA_REF * r - _REF_SCALE * s
    u_ref[...] = u
    r_ref[...] = r


def _slayer_layer_pallas(x_tbc, w_t, *, bo, tc, nb=2, out_dtype=jnp.bfloat16):
    t_dim, b, cin = x_tbc.shape
    cout = w_t.shape[1]
    bh = b // nb
    grid = (nb, cout // bo, t_dim // tc)
    return pl.pallas_call(
        functools.partial(_layer_body, tc=tc, bh=bh),
        grid=grid,
        in_specs=[
            pl.BlockSpec((tc, bh, cin), lambda i, j, k: (k, i, 0)),
            pl.BlockSpec((cin, bo), lambda i, j, k: (0, j)),
        ],
        out_specs=pl.BlockSpec((tc, bh, bo), lambda i, j, k: (k, i, j)),
        out_shape=jax.ShapeDtypeStruct((t_dim, b, cout), out_dtype),
        scratch_shapes=[
            pltpu.VMEM((bh, bo), jnp.float32),
            pltpu.VMEM((bh, bo), jnp.float32),
            pltpu.VMEM((tc * bh, bo), jnp.float32),
        ],
        compiler_params=pltpu.CompilerParams(
            dimension_semantics=("parallel", "parallel", "arbitrary"),
            vmem_limit_bytes=56 * 1024 * 1024,
        ),
        name="slayer_layer",
    )(x_tbc, w_t)


def kernel(spike_input, W1, W2):
    x = jnp.transpose(spike_input.astype(jnp.int8), (2, 0, 1))
    s1 = _slayer_layer_pallas(x, W1.T.astype(jnp.bfloat16),
                              bo=1024, tc=25, nb=1, out_dtype=jnp.int8)
    s2 = _slayer_layer_pallas(s1, W2.T.astype(jnp.bfloat16),
                              bo=512, tc=25, nb=1, out_dtype=jnp.int8)
    return jnp.transpose(s2, (1, 2, 0)).astype(jnp.float32)
